# baseline (speedup 1.0000x reference)
import functools

import jax
import jax.numpy as jnp
from jax import lax
from jax.experimental import pallas as pl
from jax.experimental.pallas import tpu as pltpu
from jax.experimental.pallas import tpu_sc as plsc

N = 1_600_000
B = 32
K = 64
NSEG = B * K
ROW = 2 * NSEG
NW = 32
ROWS_PER_W = N // NW
CHUNK = 2000
NCHUNK = ROWS_PER_W // CHUNK
VPC = CHUNK // 16
LSTRIDE = NSEG

PBLK = 131072
PGRID = -(-N // PBLK)


def _prepass(reco_ref, inp_ref, cl_ref, sq_ref, seg_ref):
    d = reco_ref[...] - inp_ref[4:5, :]
    seg = cl_ref[3:4, :] * K + cl_ref[4:5, :]
    sq_ref[...] = jnp.reshape(d * d, (PBLK,))
    seg_ref[...] = jnp.reshape(seg, (PBLK,))


_mesh = plsc.VectorSubcoreMesh(core_axis_name="c", subcore_axis_name="s")


@functools.partial(
    pl.kernel,
    mesh=_mesh,
    out_type=jax.ShapeDtypeStruct((NW, ROW), jnp.float32),
    compiler_params=pltpu.CompilerParams(needs_layout_passes=False),
    scratch_types=[
        pltpu.VMEM((CHUNK + 32,), jnp.float32),
        pltpu.VMEM((CHUNK + 32,), jnp.float32),
        pltpu.VMEM((CHUNK + 32,), jnp.int32),
        pltpu.VMEM((CHUNK + 32,), jnp.int32),
        pltpu.VMEM((16 * LSTRIDE,), jnp.float32),
        pltpu.VMEM((16 * LSTRIDE,), jnp.float32),
        pltpu.SemaphoreType.DMA,
        pltpu.SemaphoreType.DMA,
    ],
)
def _seg_reduce(sq_h, seg_h, out_h, sq_v0, sq_v1, seg_v0, seg_v1,
                acc_s, acc_c, sem0, sem1):
    c = lax.axis_index("c")
    s = lax.axis_index("s")
    wid = c * 16 + s
    base = wid * ROWS_PER_W

    zeros = jnp.zeros((16,), jnp.float32)
    ones = jnp.ones((16,), jnp.float32)
    lane = lax.iota(jnp.int32, 16)
    lane_base = lane * LSTRIDE
    sems = (sem0, sem1)
    sq_bufs = (sq_v0, sq_v1)
    seg_bufs = (seg_v0, seg_v1)

    def start(ch):
        b = ch % 2
        r0 = base + ch * CHUNK
        h1 = pltpu.async_copy(
            sq_h.at[pl.ds(r0, CHUNK)], sq_bufs[b].at[pl.ds(0, CHUNK)], sems[b])
        h2 = pltpu.async_copy(
            seg_h.at[pl.ds(r0, CHUNK)], seg_bufs[b].at[pl.ds(0, CHUNK)], sems[b])
        return h1, h2

    hs = start(0)

    def zbody(i, carry):
        p = pl.multiple_of(i * 16, 16)
        acc_s[pl.ds(p, 16)] = zeros
        acc_c[pl.ds(p, 16)] = zeros
        return carry

    lax.fori_loop(0, 16 * LSTRIDE // 16, zbody, 0)

    U = 25
    for ch in range(NCHUNK):
        h1, h2 = hs
        h1.wait()
        h2.wait()
        if ch + 1 < NCHUNK:
            hs = start(ch + 1)
        sq_v = sq_bufs[ch % 2]
        seg_v = seg_bufs[ch % 2]

        def ld(p):
            p = pl.multiple_of(p, 16)
            return sq_v[pl.ds(p, 16)], seg_v[pl.ds(p, 16)]

        car0 = ld(0) + ld(16)

        def vbody(i, carry2):
            sq_a, seg_a, sq_b, seg_b = carry2
            for u in range(U):
                p2 = pl.multiple_of(i * (16 * U) + (u + 2) * 16, 16)
                sq_n = sq_v[pl.ds(p2, 16)]
                seg_n = seg_v[pl.ds(p2, 16)]
                idx = lane_base + seg_a
                plsc.addupdate_scatter(acc_s, [idx], sq_a)
                plsc.addupdate_scatter(acc_c, [idx], ones)
                sq_a, seg_a = sq_b, seg_b
                sq_b, seg_b = sq_n, seg_n
            return sq_a, seg_a, sq_b, seg_b

        lax.fori_loop(0, VPC // U, vbody, car0)

    def rbody(j, carry):
        p = pl.multiple_of(j * 16, 16)
        for acc in (acc_s, acc_c):
            v = [acc[pl.ds(l * LSTRIDE + p, 16)] for l in range(16)]
            while len(v) > 1:
                v = [a + b for a, b in zip(v[::2], v[1::2])]
            acc[pl.ds(p, 16)] = v[0]
        return carry

    lax.fori_loop(0, NSEG // 16, rbody, 0)

    pltpu.sync_copy(acc_s.at[pl.ds(0, NSEG)], out_h.at[wid, pl.ds(0, NSEG)])
    pltpu.sync_copy(acc_c.at[pl.ds(0, NSEG)], out_h.at[wid, pl.ds(NSEG, NSEG)])


def _epilogue(p_ref, o_ref):
    p = p_ref[...]
    s = jnp.sum(p[:, :NSEG], axis=0, keepdims=True)
    cnt = jnp.sum(p[:, NSEG:], axis=0, keepdims=True)
    pres = cnt > 0.0
    mse = jnp.where(pres, s / jnp.maximum(cnt, 1.0), 0.0)
    a = (lax.broadcasted_iota(jnp.int32, (NSEG, B), 0) // K
         == lax.broadcasted_iota(jnp.int32, (NSEG, B), 1)
         ).astype(jnp.float32)
    bsum = jnp.dot(mse, a, preferred_element_type=jnp.float32,
                   precision=lax.Precision.HIGHEST)
    ncl = jnp.dot(pres.astype(jnp.float32), a,
                  preferred_element_type=jnp.float32,
                  precision=lax.Precision.HIGHEST)
    bl = bsum / jnp.maximum(ncl, 1.0)
    bp = ncl > 0.0
    loss = jnp.sum(jnp.where(bp, bl, 0.0)) / jnp.maximum(
        jnp.sum(bp.astype(jnp.float32)), 1.0)
    o_ref[...] = jnp.full((1, 1), loss, jnp.float32)


def kernel(reco, input_data0, cluster_label0):
    sq, seg = pl.pallas_call(
        _prepass,
        grid=(PGRID,),
        in_specs=[
            pl.BlockSpec((1, PBLK), lambda i: (0, i)),
            pl.BlockSpec((5, PBLK), lambda i: (0, i)),
            pl.BlockSpec((6, PBLK), lambda i: (0, i)),
        ],
        out_specs=[
            pl.BlockSpec((PBLK,), lambda i: (i,)),
            pl.BlockSpec((PBLK,), lambda i: (i,)),
        ],
        out_shape=[
            jax.ShapeDtypeStruct((N,), jnp.float32),
            jax.ShapeDtypeStruct((N,), jnp.int32),
        ],
    )(reco.T, input_data0.T, cluster_label0.T)

    parts = _seg_reduce(sq, seg)

    loss2d = pl.pallas_call(
        _epilogue,
        out_shape=jax.ShapeDtypeStruct((1, 1), jnp.float32),
    )(parts)
    return loss2d[0, 0]


# speedup vs baseline: 1.1469x; 1.1469x over previous
import functools

import jax
import jax.numpy as jnp
from jax import lax
from jax.experimental import pallas as pl
from jax.experimental.pallas import tpu as pltpu
from jax.experimental.pallas import tpu_sc as plsc

N = 1_600_000
B = 32
K = 64
NSEG = B * K
ROW = 2 * NSEG
NW = 32
ROWS_PER_W = N // NW
CHUNK = 10000
NCHUNK = ROWS_PER_W // CHUNK
VPC = CHUNK // 16
LSTRIDE = NSEG

PBLK = 131072
PGRID = -(-N // PBLK)


def _prepass(reco_ref, inp_ref, cl_ref, sq_ref, seg_ref):
    d = reco_ref[...] - inp_ref[4:5, :]
    seg = cl_ref[3:4, :] * K + cl_ref[4:5, :]
    sq_ref[...] = jnp.reshape(d * d, (PBLK,))
    seg_ref[...] = jnp.reshape(seg, (PBLK,))


_mesh = plsc.VectorSubcoreMesh(core_axis_name="c", subcore_axis_name="s")


@functools.partial(
    pl.kernel,
    mesh=_mesh,
    out_type=jax.ShapeDtypeStruct((NW, ROW), jnp.float32),
    compiler_params=pltpu.CompilerParams(needs_layout_passes=False),
    scratch_types=[
        pltpu.VMEM((CHUNK + 32,), jnp.float32),
        pltpu.VMEM((CHUNK + 32,), jnp.float32),
        pltpu.VMEM((CHUNK + 32,), jnp.int32),
        pltpu.VMEM((CHUNK + 32,), jnp.int32),
        pltpu.VMEM((16 * LSTRIDE,), jnp.float32),
        pltpu.VMEM((16 * LSTRIDE,), jnp.float32),
        pltpu.SemaphoreType.DMA,
        pltpu.SemaphoreType.DMA,
    ],
)
def _seg_reduce(sq_h, seg_h, out_h, sq_v0, sq_v1, seg_v0, seg_v1,
                acc_s, acc_c, sem0, sem1):
    c = lax.axis_index("c")
    s = lax.axis_index("s")
    wid = c * 16 + s
    base = wid * ROWS_PER_W

    zeros = jnp.zeros((16,), jnp.float32)
    ones = jnp.ones((16,), jnp.float32)
    lane = lax.iota(jnp.int32, 16)
    lane_base = lane * LSTRIDE
    sems = (sem0, sem1)
    sq_bufs = (sq_v0, sq_v1)
    seg_bufs = (seg_v0, seg_v1)

    def start(ch):
        b = ch % 2
        r0 = base + ch * CHUNK
        h1 = pltpu.async_copy(
            sq_h.at[pl.ds(r0, CHUNK)], sq_bufs[b].at[pl.ds(0, CHUNK)], sems[b])
        h2 = pltpu.async_copy(
            seg_h.at[pl.ds(r0, CHUNK)], seg_bufs[b].at[pl.ds(0, CHUNK)], sems[b])
        return h1, h2

    hs = start(0)

    def zbody(i, carry):
        p = pl.multiple_of(i * 16, 16)
        acc_s[pl.ds(p, 16)] = zeros
        acc_c[pl.ds(p, 16)] = zeros
        return carry

    lax.fori_loop(0, 16 * LSTRIDE // 16, zbody, 0)

    U = 25
    for ch in range(NCHUNK):
        h1, h2 = hs
        h1.wait()
        h2.wait()
        if ch + 1 < NCHUNK:
            hs = start(ch + 1)
        sq_v = sq_bufs[ch % 2]
        seg_v = seg_bufs[ch % 2]

        def ld(p):
            p = pl.multiple_of(p, 16)
            return sq_v[pl.ds(p, 16)], seg_v[pl.ds(p, 16)]

        car0 = ld(0) + ld(16)

        def vbody(i, carry2):
            sq_a, seg_a, sq_b, seg_b = carry2
            for u in range(U):
                p2 = pl.multiple_of(i * (16 * U) + (u + 2) * 16, 16)
                sq_n = sq_v[pl.ds(p2, 16)]
                seg_n = seg_v[pl.ds(p2, 16)]
                idx = lane_base + seg_a
                plsc.addupdate_scatter(acc_s, [idx], sq_a)
                plsc.addupdate_scatter(acc_c, [idx], ones)
                sq_a, seg_a = sq_b, seg_b
                sq_b, seg_b = sq_n, seg_n
            return sq_a, seg_a, sq_b, seg_b

        lax.fori_loop(0, VPC // U, vbody, car0)

    def rbody(j, carry):
        p = pl.multiple_of(j * 16, 16)
        for acc in (acc_s, acc_c):
            v = [acc[pl.ds(l * LSTRIDE + p, 16)] for l in range(16)]
            while len(v) > 1:
                v = [a + b for a, b in zip(v[::2], v[1::2])]
            acc[pl.ds(p, 16)] = v[0]
        return carry

    lax.fori_loop(0, NSEG // 16, rbody, 0)

    pltpu.sync_copy(acc_s.at[pl.ds(0, NSEG)], out_h.at[wid, pl.ds(0, NSEG)])
    pltpu.sync_copy(acc_c.at[pl.ds(0, NSEG)], out_h.at[wid, pl.ds(NSEG, NSEG)])


def _epilogue(p_ref, o_ref):
    p = p_ref[...]
    s = jnp.sum(p[:, :NSEG], axis=0, keepdims=True)
    cnt = jnp.sum(p[:, NSEG:], axis=0, keepdims=True)
    pres = cnt > 0.0
    mse = jnp.where(pres, s / jnp.maximum(cnt, 1.0), 0.0)
    a = (lax.broadcasted_iota(jnp.int32, (NSEG, B), 0) // K
         == lax.broadcasted_iota(jnp.int32, (NSEG, B), 1)
         ).astype(jnp.float32)
    bsum = jnp.dot(mse, a, preferred_element_type=jnp.float32,
                   precision=lax.Precision.HIGHEST)
    ncl = jnp.dot(pres.astype(jnp.float32), a,
                  preferred_element_type=jnp.float32,
                  precision=lax.Precision.HIGHEST)
    bl = bsum / jnp.maximum(ncl, 1.0)
    bp = ncl > 0.0
    loss = jnp.sum(jnp.where(bp, bl, 0.0)) / jnp.maximum(
        jnp.sum(bp.astype(jnp.float32)), 1.0)
    o_ref[...] = jnp.full((1, 1), loss, jnp.float32)


def kernel(reco, input_data0, cluster_label0):
    sq, seg = pl.pallas_call(
        _prepass,
        grid=(PGRID,),
        in_specs=[
            pl.BlockSpec((1, PBLK), lambda i: (0, i)),
            pl.BlockSpec((5, PBLK), lambda i: (0, i)),
            pl.BlockSpec((6, PBLK), lambda i: (0, i)),
        ],
        out_specs=[
            pl.BlockSpec((PBLK,), lambda i: (i,)),
            pl.BlockSpec((PBLK,), lambda i: (i,)),
        ],
        out_shape=[
            jax.ShapeDtypeStruct((N,), jnp.float32),
            jax.ShapeDtypeStruct((N,), jnp.int32),
        ],
    )(reco.T, input_data0.T, cluster_label0.T)

    parts = _seg_reduce(sq, seg)

    loss2d = pl.pallas_call(
        _epilogue,
        out_shape=jax.ShapeDtypeStruct((1, 1), jnp.float32),
    )(parts)
    return loss2d[0, 0]
